# Initial kernel scaffold; baseline (speedup 1.0000x reference)
import jax
import jax.numpy as jnp
from jax.experimental import pallas as pl


def kernel(x, edge_index, W1, b1, W2, b2, Wc1, bc1, Wc2, bc2, Wc3, bc3):
    raise NotImplementedError("write your pallas kernel here")


# speedup vs baseline: 7.3734x; 7.3734x over previous
import functools

import jax
import jax.numpy as jnp
from jax import lax
from jax.experimental import pallas as pl
from jax.experimental.pallas import tpu as pltpu
from jax.experimental.pallas import tpu_sc as plsc

N = 10000
E = 320000
D = 128
H = 128
C = 1

NC = 2
NS = 16
NW = NC * NS
LANES = 16
EPT = E // NW
CH = 80
NCH = EPT // CH
RPT = N // NS
ZCH = 125
NZ = RPT // ZCH

_F32 = jnp.float32
_HI = lax.Precision.HIGHEST


def _deg_body(src_hbm, dst_hbm, out_hbm, idx_s, idx_d, degs, degd):
    c = lax.axis_index("c")
    s = lax.axis_index("s")
    wid = s * NC + c
    pltpu.sync_copy(src_hbm.at[pl.ds(wid * NCH, NCH)], idx_s)
    pltpu.sync_copy(dst_hbm.at[pl.ds(wid * NCH, NCH)], idx_d)
    zeros16 = jnp.zeros((LANES,), _F32)

    def zbody(i, carry):
        degs[pl.ds(i * LANES, LANES)] = zeros16
        degd[pl.ds(i * LANES, LANES)] = zeros16
        return carry

    lax.fori_loop(0, N // LANES, zbody, 0)
    ones16 = jnp.ones((LANES,), _F32)

    def hbody(i, carry):
        for j in range(CH // LANES):
            v = idx_s[i, pl.ds(j * LANES, LANES)]
            plsc.addupdate_scatter(degs, [v], ones16)
            w = idx_d[i, pl.ds(j * LANES, LANES)]
            plsc.addupdate_scatter(degd, [w], ones16)
        return carry

    lax.fori_loop(0, NCH, hbody, 0)
    pltpu.sync_copy(degs, out_hbm.at[wid, 0])
    pltpu.sync_copy(degd, out_hbm.at[wid, 1])


@functools.cache
def _deg():
    mesh = plsc.VectorSubcoreMesh(core_axis_name="c", subcore_axis_name="s",
                                  num_cores=NC, num_subcores=NS)
    return pl.kernel(
        _deg_body,
        out_type=jax.ShapeDtypeStruct((NW, 2, N), _F32),
        mesh=mesh,
        compiler_params=pltpu.CompilerParams(use_tc_tiling_on_sc=False,
                                             needs_layout_passes=False),
        scratch_types=[
            pltpu.VMEM((NCH, CH), jnp.int32),
            pltpu.VMEM((NCH, CH), jnp.int32),
            pltpu.VMEM((N,), _F32),
            pltpu.VMEM((N,), _F32),
        ],
    )


def _mp_body(t_hbm, src_hbm, dst_hbm, out_hbm, agg_sh, idx_s, idx_d, rows,
             zbuf, sem):
    c = lax.axis_index("c")
    s = lax.axis_index("s")
    wid = s * NC + c
    pltpu.sync_copy(src_hbm.at[pl.ds(wid * NCH, NCH)], idx_s)
    pltpu.sync_copy(dst_hbm.at[pl.ds(wid * NCH, NCH)], idx_d)
    zeros16 = jnp.zeros((LANES,), _F32)

    def zb(i, carry):
        for j in range(H // LANES):
            zbuf[i, pl.ds(j * LANES, LANES)] = zeros16
        return carry

    lax.fori_loop(0, ZCH, zb, 0)
    for k in range(NZ):
        pltpu.sync_copy(zbuf, agg_sh.at[pl.ds(s * RPT + k * ZCH, ZCH)])
    plsc.subcore_barrier()

    def mp(j, carry):
        pltpu.async_copy(t_hbm.at[idx_s.at[j]], rows, sem).wait()
        pltpu.sync_copy(rows, agg_sh.at[idx_d.at[j]], add=True)
        return carry

    lax.fori_loop(0, NCH, mp, 0)
    plsc.subcore_barrier()
    for k in range(NZ):
        pltpu.sync_copy(agg_sh.at[pl.ds(s * RPT + k * ZCH, ZCH)], zbuf)
        pltpu.sync_copy(zbuf, out_hbm.at[c, pl.ds(s * RPT + k * ZCH, ZCH)])


@functools.cache
def _mp():
    mesh = plsc.VectorSubcoreMesh(core_axis_name="c", subcore_axis_name="s",
                                  num_cores=NC, num_subcores=NS)
    return pl.kernel(
        _mp_body,
        out_type=jax.ShapeDtypeStruct((NC, N, H), _F32),
        mesh=mesh,
        compiler_params=pltpu.CompilerParams(use_tc_tiling_on_sc=False,
                                             needs_layout_passes=False),
        scratch_types=[
            pltpu.VMEM_SHARED((N, H), _F32),
            pltpu.VMEM((NCH, CH), jnp.int32),
            pltpu.VMEM((NCH, CH), jnp.int32),
            pltpu.VMEM((CH, H), _F32),
            pltpu.VMEM((ZCH, H), _F32),
            pltpu.SemaphoreType.DMA,
        ],
    )


def _norm_body(degp_ref, norm_ref):
    acc = degp_ref[0]
    for i in range(1, NW):
        acc = acc + degp_ref[i]
    norm_ref[...] = lax.rsqrt(jnp.maximum(acc, 1.0))


_norms = pl.pallas_call(
    _norm_body,
    out_shape=jax.ShapeDtypeStruct((2, N), _F32),
)


def _mm1_body(x_ref, w_ref, ns_ref, o_ref):
    o_ref[...] = jnp.dot(x_ref[...], w_ref[...], preferred_element_type=_F32,
                         precision=_HI) * ns_ref[...]


_mm1 = pl.pallas_call(
    _mm1_body,
    out_shape=jax.ShapeDtypeStruct((N, H), _F32),
)


def _tcb_body(p_ref, nd_ref, b1_ref, w2_ref, ns_ref, o_ref):
    h = jnp.maximum((p_ref[0] + p_ref[1]) * nd_ref[...] + b1_ref[...], 0.0)
    o_ref[...] = jnp.dot(h, w2_ref[...], preferred_element_type=_F32,
                         precision=_HI) * ns_ref[...]


_tcb = pl.pallas_call(
    _tcb_body,
    out_shape=jax.ShapeDtypeStruct((N, H), _F32),
)


def _tcc_body(q_ref, nd_ref, b2_ref, wc1_ref, bc1_ref, wc2_ref, bc2_ref,
              wc3_ref, bc3_ref, o_ref):
    h2 = jnp.maximum((q_ref[0] + q_ref[1]) * nd_ref[...] + b2_ref[...], 0.0)
    hg = jnp.mean(h2, axis=0, keepdims=True)
    o1 = jnp.maximum(jnp.dot(hg, wc1_ref[...], preferred_element_type=_F32,
                             precision=_HI) + bc1_ref[...], 0.0)
    o2 = jnp.maximum(jnp.dot(o1, wc2_ref[...], preferred_element_type=_F32,
                             precision=_HI) + bc2_ref[...], 0.0)
    o_ref[...] = jnp.dot(o2, wc3_ref[...], preferred_element_type=_F32,
                         precision=_HI) + bc3_ref[...]


_tcc = pl.pallas_call(
    _tcc_body,
    out_shape=jax.ShapeDtypeStruct((1, C), _F32),
)


def kernel(x, edge_index, W1, b1, W2, b2, Wc1, bc1, Wc2, bc2, Wc3, bc3):
    src2d = edge_index[0].reshape(E // CH, CH)
    dst2d = edge_index[1].reshape(E // CH, CH)
    degp = _deg()(src2d, dst2d)
    norms = _norms(degp)
    ns = norms[0].reshape(N, 1)
    nd = norms[1].reshape(N, 1)
    t1 = _mm1(x, W1, ns)
    p = _mp()(t1, src2d, dst2d)
    t2 = _tcb(p, nd, b1.reshape(1, H), W2, ns)
    q = _mp()(t2, src2d, dst2d)
    o = _tcc(q, nd, b2.reshape(1, H), Wc1, bc1.reshape(1, H), Wc2,
             bc2.reshape(1, H), Wc3, bc3.reshape(1, C))
    return o


# speedup vs baseline: 11.4245x; 1.5494x over previous
import functools

import jax
import jax.numpy as jnp
from jax import lax
from jax.experimental import pallas as pl
from jax.experimental.pallas import tpu as pltpu
from jax.experimental.pallas import tpu_sc as plsc

N = 10000
E = 320000
D = 128
H = 128
HH = H // 2
C = 1

NC = 2
NS = 16
NW = NC * NS
LANES = 16

DEPT = E // NW
DCH = 80
DNCH = DEPT // DCH

EPT = E // NS
CH = 125
NCH = EPT // CH
R = 5
NG = NCH // R
RPT = N // NS
ZCH = 125
NZ = RPT // ZCH

_F32 = jnp.float32
_HI = lax.Precision.HIGHEST


def _deg_body(src_hbm, dst_hbm, out_hbm, idx_s, idx_d, degs, degd):
    c = lax.axis_index("c")
    s = lax.axis_index("s")
    wid = s * NC + c
    pltpu.sync_copy(src_hbm.at[pl.ds(wid * DNCH, DNCH)], idx_s)
    pltpu.sync_copy(dst_hbm.at[pl.ds(wid * DNCH, DNCH)], idx_d)
    zeros16 = jnp.zeros((LANES,), _F32)

    def zbody(i, carry):
        degs[pl.ds(i * LANES, LANES)] = zeros16
        degd[pl.ds(i * LANES, LANES)] = zeros16
        return carry

    lax.fori_loop(0, N // LANES, zbody, 0)
    ones16 = jnp.ones((LANES,), _F32)

    def hbody(i, carry):
        for j in range(DCH // LANES):
            v = idx_s[i, pl.ds(j * LANES, LANES)]
            plsc.addupdate_scatter(degs, [v], ones16)
            w = idx_d[i, pl.ds(j * LANES, LANES)]
            plsc.addupdate_scatter(degd, [w], ones16)
        return carry

    lax.fori_loop(0, DNCH, hbody, 0)
    pltpu.sync_copy(degs, out_hbm.at[wid, 0])
    pltpu.sync_copy(degd, out_hbm.at[wid, 1])


@functools.cache
def _deg():
    mesh = plsc.VectorSubcoreMesh(core_axis_name="c", subcore_axis_name="s",
                                  num_cores=NC, num_subcores=NS)
    return pl.kernel(
        _deg_body,
        out_type=jax.ShapeDtypeStruct((NW, 2, N), _F32),
        mesh=mesh,
        compiler_params=pltpu.CompilerParams(use_tc_tiling_on_sc=False,
                                             needs_layout_passes=False),
        scratch_types=[
            pltpu.VMEM((DNCH, DCH), jnp.int32),
            pltpu.VMEM((DNCH, DCH), jnp.int32),
            pltpu.VMEM((N,), _F32),
            pltpu.VMEM((N,), _F32),
        ],
    )


def _mp_body(t_hbm, src_hbm, dst_hbm, out_hbm, agg_sh, idx_s, idx_d, rows,
             gsem, ssem):
    c = lax.axis_index("c")
    s = lax.axis_index("s")
    tc = t_hbm.at[c]
    pltpu.sync_copy(src_hbm.at[pl.ds(s * NCH, NCH)], idx_s)
    pltpu.sync_copy(dst_hbm.at[pl.ds(s * NCH, NCH)], idx_d)
    zeros16 = jnp.zeros((LANES,), _F32)

    def zb(i, carry):
        for j in range(HH // LANES):
            rows[0, i, pl.ds(j * LANES, LANES)] = zeros16
        return carry

    lax.fori_loop(0, ZCH, zb, 0)
    for k in range(NZ):
        pltpu.sync_copy(rows.at[0], agg_sh.at[pl.ds(s * RPT + k * ZCH, ZCH)])
    plsc.subcore_barrier()

    def g_issue(j, b):
        pltpu.async_copy(tc.at[idx_s.at[j]], rows.at[b], gsem.at[b])

    def g_wait(b):
        pltpu.make_async_copy(tc.at[idx_s.at[0]], rows.at[b],
                              gsem.at[b]).wait()

    def s_issue(j, b):
        pltpu.async_copy(rows.at[b], agg_sh.at[idx_d.at[j]], ssem.at[b],
                         add=True)

    def s_wait(b):
        pltpu.make_async_copy(rows.at[b], agg_sh.at[idx_d.at[0]],
                              ssem.at[b]).wait()

    for b in range(R - 1):
        g_issue(b, b)
    for b in range(R):
        g_wait(b)
        s_issue(b, b)
        if b >= 1:
            s_wait((b - 1) % R)
        g_issue(b + R - 1, (b - 1) % R)

    def grp(g, carry):
        for b in range(R):
            j = g * R + b
            g_wait(b)
            s_issue(j, b)
            s_wait((b - 1) % R)
            g_issue(j + R - 1, (b - 1) % R)
        return carry

    lax.fori_loop(1, NG - 1, grp, 0)
    for b in range(R):
        j = (NG - 1) * R + b
        g_wait(b)
        s_issue(j, b)
        if b == 0:
            s_wait((b - 1) % R)
            g_issue(j + R - 1, (b - 1) % R)
    for b in range(R):
        s_wait(b)

    plsc.subcore_barrier()
    for k in range(NZ):
        pltpu.sync_copy(agg_sh.at[pl.ds(s * RPT + k * ZCH, ZCH)], rows.at[0])
        pltpu.sync_copy(rows.at[0],
                        out_hbm.at[c, pl.ds(s * RPT + k * ZCH, ZCH)])


@functools.cache
def _mp():
    mesh = plsc.VectorSubcoreMesh(core_axis_name="c", subcore_axis_name="s",
                                  num_cores=NC, num_subcores=NS)
    return pl.kernel(
        _mp_body,
        out_type=jax.ShapeDtypeStruct((NC, N, HH), _F32),
        mesh=mesh,
        compiler_params=pltpu.CompilerParams(use_tc_tiling_on_sc=False,
                                             needs_layout_passes=False),
        scratch_types=[
            pltpu.VMEM_SHARED((N, HH), _F32),
            pltpu.VMEM((NCH, CH), jnp.int32),
            pltpu.VMEM((NCH, CH), jnp.int32),
            pltpu.VMEM((R, CH, HH), _F32),
            pltpu.SemaphoreType.DMA((R,)),
            pltpu.SemaphoreType.DMA((R,)),
        ],
    )


def _norm_body(degp_ref, norm_ref):
    acc = degp_ref[0]
    for i in range(1, NW):
        acc = acc + degp_ref[i]
    norm_ref[...] = lax.rsqrt(jnp.maximum(acc, 1.0))


_norms = pl.pallas_call(
    _norm_body,
    out_shape=jax.ShapeDtypeStruct((2, N), _F32),
)


def _mm1_body(x_ref, w_ref, ns_ref, o_ref):
    for c in range(NC):
        o_ref[c] = jnp.dot(x_ref[...], w_ref[c], preferred_element_type=_F32,
                           precision=_HI) * ns_ref[...]


_mm1 = pl.pallas_call(
    _mm1_body,
    out_shape=jax.ShapeDtypeStruct((NC, N, HH), _F32),
)


def _tcb_body(p_ref, nd_ref, b1_ref, w2_ref, ns_ref, o_ref):
    h0 = jnp.maximum(p_ref[0] * nd_ref[...] + b1_ref[0], 0.0)
    h1 = jnp.maximum(p_ref[1] * nd_ref[...] + b1_ref[1], 0.0)
    for c in range(NC):
        t2 = (jnp.dot(h0, w2_ref[0, c], preferred_element_type=_F32,
                      precision=_HI)
              + jnp.dot(h1, w2_ref[1, c], preferred_element_type=_F32,
                        precision=_HI))
        o_ref[c] = t2 * ns_ref[...]


_MB = 2000

_tcb = pl.pallas_call(
    _tcb_body,
    grid=(N // _MB,),
    in_specs=[
        pl.BlockSpec((NC, _MB, HH), lambda i: (0, i, 0)),
        pl.BlockSpec((_MB, 1), lambda i: (i, 0)),
        pl.BlockSpec((NC, 1, HH), lambda i: (0, 0, 0)),
        pl.BlockSpec((NC, NC, HH, HH), lambda i: (0, 0, 0, 0)),
        pl.BlockSpec((_MB, 1), lambda i: (i, 0)),
    ],
    out_specs=pl.BlockSpec((NC, _MB, HH), lambda i: (0, i, 0)),
    out_shape=jax.ShapeDtypeStruct((NC, N, HH), _F32),
)


def _tcc_body(q_ref, nd_ref, b2_ref, wc1_ref, bc1_ref, wc2_ref, bc2_ref,
              wc3_ref, bc3_ref, o_ref):
    hg = []
    for c in range(NC):
        h2 = jnp.maximum(q_ref[c] * nd_ref[...] + b2_ref[c], 0.0)
        hg.append(jnp.mean(h2, axis=0, keepdims=True))
    o1 = jnp.maximum(jnp.dot(hg[0], wc1_ref[0], preferred_element_type=_F32,
                             precision=_HI)
                     + jnp.dot(hg[1], wc1_ref[1], preferred_element_type=_F32,
                               precision=_HI)
                     + bc1_ref[...], 0.0)
    o2 = jnp.maximum(jnp.dot(o1, wc2_ref[...], preferred_element_type=_F32,
                             precision=_HI) + bc2_ref[...], 0.0)
    o_ref[...] = jnp.dot(o2, wc3_ref[...], preferred_element_type=_F32,
                         precision=_HI) + bc3_ref[...]


_tcc = pl.pallas_call(
    _tcc_body,
    out_shape=jax.ShapeDtypeStruct((1, C), _F32),
)


def _split_cols(w):
    return jnp.stack([w[:, :HH], w[:, HH:]])


def _split_rows(w):
    return jnp.stack([w[:HH, :], w[HH:, :]])


def kernel(x, edge_index, W1, b1, W2, b2, Wc1, bc1, Wc2, bc2, Wc3, bc3):
    srcd = edge_index[0].reshape(E // DCH, DCH)
    dstd = edge_index[1].reshape(E // DCH, DCH)
    srcm = edge_index[0].reshape(E // CH, CH)
    dstm = edge_index[1].reshape(E // CH, CH)
    degp = _deg()(srcd, dstd)
    norms = _norms(degp)
    ns = norms[0].reshape(N, 1)
    nd = norms[1].reshape(N, 1)
    w1s = _split_cols(W1)
    w2q = jnp.stack([_split_cols(W2[:HH]), _split_cols(W2[HH:])])
    b1s = b1.reshape(2, 1, HH)
    b2s = b2.reshape(2, 1, HH)
    wc1s = _split_rows(Wc1)
    t1 = _mm1(x, w1s, ns)
    p = _mp()(t1, srcm, dstm)
    t2 = _tcb(p, nd, b1s, w2q, ns)
    q = _mp()(t2, srcm, dstm)
    o = _tcc(q, nd, b2s, wc1s, bc1.reshape(1, H), Wc2, bc2.reshape(1, H),
             Wc3, bc3.reshape(1, C))
    return o


# speedup vs baseline: 14.7194x; 1.2884x over previous
import functools

import jax
import jax.numpy as jnp
from jax import lax
from jax.experimental import pallas as pl
from jax.experimental.pallas import tpu as pltpu
from jax.experimental.pallas import tpu_sc as plsc

N = 10000
E = 320000
D = 128
H = 128
HH = H // 2
C = 1

NC = 2
NS = 16
NW = NC * NS
LANES = 16

DEPT = E // NW
DCH = 80
DNCH = DEPT // DCH

EPT = E // NS
CH = 125
NCH = EPT // CH
R = 8
NG = NCH // R
RPT = N // NS
ZCH = 125
NZ = RPT // ZCH

_F32 = jnp.float32
_BF16 = jnp.bfloat16
_HI = lax.Precision.HIGHEST


def _deg_body(src_hbm, dst_hbm, out_hbm, idx_s, idx_d, degs, degd):
    c = lax.axis_index("c")
    s = lax.axis_index("s")
    wid = s * NC + c
    pltpu.sync_copy(src_hbm.at[pl.ds(wid * DNCH, DNCH)], idx_s)
    pltpu.sync_copy(dst_hbm.at[pl.ds(wid * DNCH, DNCH)], idx_d)
    zeros16 = jnp.zeros((LANES,), _F32)

    def zbody(i, carry):
        degs[pl.ds(i * LANES, LANES)] = zeros16
        degd[pl.ds(i * LANES, LANES)] = zeros16
        return carry

    lax.fori_loop(0, N // LANES, zbody, 0)
    ones16 = jnp.ones((LANES,), _F32)

    def hbody(i, carry):
        for j in range(DCH // LANES):
            v = idx_s[i, pl.ds(j * LANES, LANES)]
            plsc.addupdate_scatter(degs, [v], ones16)
            w = idx_d[i, pl.ds(j * LANES, LANES)]
            plsc.addupdate_scatter(degd, [w], ones16)
        return carry

    lax.fori_loop(0, DNCH, hbody, 0)
    pltpu.sync_copy(degs, out_hbm.at[wid, 0])
    pltpu.sync_copy(degd, out_hbm.at[wid, 1])


@functools.cache
def _deg():
    mesh = plsc.VectorSubcoreMesh(core_axis_name="c", subcore_axis_name="s",
                                  num_cores=NC, num_subcores=NS)
    return pl.kernel(
        _deg_body,
        out_type=jax.ShapeDtypeStruct((NW, 2, N), _F32),
        mesh=mesh,
        compiler_params=pltpu.CompilerParams(use_tc_tiling_on_sc=False,
                                             needs_layout_passes=False),
        scratch_types=[
            pltpu.VMEM((DNCH, DCH), jnp.int32),
            pltpu.VMEM((DNCH, DCH), jnp.int32),
            pltpu.VMEM((N,), _F32),
            pltpu.VMEM((N,), _F32),
        ],
    )


def _mp_body(t_hbm, src_hbm, dst_hbm, out_hbm, agg_sh, idx_s, idx_d, rows,
             gsem, ssem):
    c = lax.axis_index("c")
    s = lax.axis_index("s")
    tc = t_hbm.at[c]
    pltpu.sync_copy(src_hbm.at[pl.ds(s * NCH, NCH)], idx_s)
    pltpu.sync_copy(dst_hbm.at[pl.ds(s * NCH, NCH)], idx_d)
    zeros32 = jnp.zeros((2 * LANES,), _BF16)

    def zb(i, carry):
        for j in range(HH // (2 * LANES)):
            rows[0, i, pl.ds(j * 2 * LANES, 2 * LANES)] = zeros32
        return carry

    lax.fori_loop(0, ZCH, zb, 0)
    for k in range(NZ):
        pltpu.sync_copy(rows.at[0], agg_sh.at[pl.ds(s * RPT + k * ZCH, ZCH)])
    plsc.subcore_barrier()

    def g_issue(j, b):
        pltpu.async_copy(tc.at[idx_s.at[j]], rows.at[b], gsem.at[b])

    def g_wait(b):
        pltpu.make_async_copy(tc.at[idx_s.at[0]], rows.at[b],
                              gsem.at[b]).wait()

    def s_issue(j, b):
        pltpu.async_copy(rows.at[b], agg_sh.at[idx_d.at[j]], ssem.at[b],
                         add=True)

    def s_wait(b):
        pltpu.make_async_copy(rows.at[b], agg_sh.at[idx_d.at[0]],
                              ssem.at[b]).wait()

    for b in range(R - 1):
        g_issue(b, b)
    for b in range(R):
        g_wait(b)
        s_issue(b, b)
        if b >= 1:
            s_wait((b - 1) % R)
        g_issue(b + R - 1, (b - 1) % R)

    def grp(g, carry):
        for b in range(R):
            j = g * R + b
            g_wait(b)
            s_issue(j, b)
            s_wait((b - 1) % R)
            g_issue(j + R - 1, (b - 1) % R)
        return carry

    lax.fori_loop(1, NG - 1, grp, 0)
    for b in range(R):
        j = (NG - 1) * R + b
        g_wait(b)
        s_issue(j, b)
        if b == 0:
            s_wait((b - 1) % R)
            g_issue(j + R - 1, (b - 1) % R)
    for b in range(R):
        s_wait(b)

    plsc.subcore_barrier()
    for k in range(NZ):
        pltpu.sync_copy(agg_sh.at[pl.ds(s * RPT + k * ZCH, ZCH)], rows.at[0])
        pltpu.sync_copy(rows.at[0],
                        out_hbm.at[c, pl.ds(s * RPT + k * ZCH, ZCH)])


@functools.cache
def _mp():
    mesh = plsc.VectorSubcoreMesh(core_axis_name="c", subcore_axis_name="s",
                                  num_cores=NC, num_subcores=NS)
    return pl.kernel(
        _mp_body,
        out_type=jax.ShapeDtypeStruct((NC, N, HH), _BF16),
        mesh=mesh,
        compiler_params=pltpu.CompilerParams(use_tc_tiling_on_sc=False,
                                             needs_layout_passes=False),
        scratch_types=[
            pltpu.VMEM_SHARED((N, HH), _BF16),
            pltpu.VMEM((NCH, CH), jnp.int32),
            pltpu.VMEM((NCH, CH), jnp.int32),
            pltpu.VMEM((R, CH, HH), _BF16),
            pltpu.SemaphoreType.DMA((R,)),
            pltpu.SemaphoreType.DMA((R,)),
        ],
    )


def _norm_body(degp_ref, norm_ref):
    acc = degp_ref[0]
    for i in range(1, NW):
        acc = acc + degp_ref[i]
    norm_ref[...] = lax.rsqrt(jnp.maximum(acc, 1.0))


_norms = pl.pallas_call(
    _norm_body,
    out_shape=jax.ShapeDtypeStruct((2, N), _F32),
)


def _mm1_body(x_ref, w_ref, ns_ref, o_ref):
    for c in range(NC):
        t = jnp.dot(x_ref[...], w_ref[c], preferred_element_type=_F32,
                    precision=_HI) * ns_ref[...]
        o_ref[c] = t.astype(_BF16)


_mm1 = pl.pallas_call(
    _mm1_body,
    out_shape=jax.ShapeDtypeStruct((NC, N, HH), _BF16),
)


def _tcb_body(p_ref, nd_ref, b1_ref, w2_ref, ns_ref, o_ref):
    h0 = jnp.maximum(p_ref[0].astype(_F32) * nd_ref[...] + b1_ref[0], 0.0)
    h1 = jnp.maximum(p_ref[1].astype(_F32) * nd_ref[...] + b1_ref[1], 0.0)
    for c in range(NC):
        t2 = (jnp.dot(h0, w2_ref[0, c], preferred_element_type=_F32,
                      precision=_HI)
              + jnp.dot(h1, w2_ref[1, c], preferred_element_type=_F32,
                        precision=_HI))
        o_ref[c] = (t2 * ns_ref[...]).astype(_BF16)


_MB = 2000

_tcb = pl.pallas_call(
    _tcb_body,
    grid=(N // _MB,),
    in_specs=[
        pl.BlockSpec((NC, _MB, HH), lambda i: (0, i, 0)),
        pl.BlockSpec((_MB, 1), lambda i: (i, 0)),
        pl.BlockSpec((NC, 1, HH), lambda i: (0, 0, 0)),
        pl.BlockSpec((NC, NC, HH, HH), lambda i: (0, 0, 0, 0)),
        pl.BlockSpec((_MB, 1), lambda i: (i, 0)),
    ],
    out_specs=pl.BlockSpec((NC, _MB, HH), lambda i: (0, i, 0)),
    out_shape=jax.ShapeDtypeStruct((NC, N, HH), _BF16),
)


def _tcc_body(q_ref, nd_ref, b2_ref, wc1_ref, bc1_ref, wc2_ref, bc2_ref,
              wc3_ref, bc3_ref, o_ref):
    hg = []
    for c in range(NC):
        h2 = jnp.maximum(q_ref[c].astype(_F32) * nd_ref[...] + b2_ref[c], 0.0)
        hg.append(jnp.mean(h2, axis=0, keepdims=True))
    o1 = jnp.maximum(jnp.dot(hg[0], wc1_ref[0], preferred_element_type=_F32,
                             precision=_HI)
                     + jnp.dot(hg[1], wc1_ref[1], preferred_element_type=_F32,
                               precision=_HI)
                     + bc1_ref[...], 0.0)
    o2 = jnp.maximum(jnp.dot(o1, wc2_ref[...], preferred_element_type=_F32,
                             precision=_HI) + bc2_ref[...], 0.0)
    o_ref[...] = jnp.dot(o2, wc3_ref[...], preferred_element_type=_F32,
                         precision=_HI) + bc3_ref[...]


_tcc = pl.pallas_call(
    _tcc_body,
    out_shape=jax.ShapeDtypeStruct((1, C), _F32),
)


def _split_cols(w):
    return jnp.stack([w[:, :HH], w[:, HH:]])


def _split_rows(w):
    return jnp.stack([w[:HH, :], w[HH:, :]])


def kernel(x, edge_index, W1, b1, W2, b2, Wc1, bc1, Wc2, bc2, Wc3, bc3):
    srcd = edge_index[0].reshape(E // DCH, DCH)
    dstd = edge_index[1].reshape(E // DCH, DCH)
    srcm = edge_index[0].reshape(E // CH, CH)
    dstm = edge_index[1].reshape(E // CH, CH)
    degp = _deg()(srcd, dstd)
    norms = _norms(degp)
    ns = norms[0].reshape(N, 1)
    nd = norms[1].reshape(N, 1)
    w1s = _split_cols(W1)
    w2q = jnp.stack([_split_cols(W2[:HH]), _split_cols(W2[HH:])])
    b1s = b1.reshape(2, 1, HH)
    b2s = b2.reshape(2, 1, HH)
    wc1s = _split_rows(Wc1)
    t1 = _mm1(x, w1s, ns)
    p = _mp()(t1, srcm, dstm)
    t2 = _tcb(p, nd, b1s, w2q, ns)
    q = _mp()(t2, srcm, dstm)
    o = _tcc(q, nd, b2s, wc1s, bc1.reshape(1, H), Wc2, bc2.reshape(1, H),
             Wc3, bc3.reshape(1, C))
    return o


# speedup vs baseline: 17.0121x; 1.1558x over previous
import functools

import jax
import jax.numpy as jnp
from jax import lax
from jax.experimental import pallas as pl
from jax.experimental.pallas import tpu as pltpu
from jax.experimental.pallas import tpu_sc as plsc

N = 10000
E = 320000
D = 128
H = 128
HH = H // 2
C = 1

NC = 2
NS = 16
NW = NC * NS
LANES = 16

DEPT = E // NW
DCH = 80
DNCH = DEPT // DCH

EPT = E // NS
CH = 125
NCH = EPT // CH
R = 8
NG = NCH // R
RPT = N // NS
ZCH = 125
NZ = RPT // ZCH

_F32 = jnp.float32
_BF16 = jnp.bfloat16
_HI = lax.Precision.HIGHEST


def _deg_body(edges_hbm, out_hbm, idx_s, idx_d, degs, degd):
    c = lax.axis_index("c")
    s = lax.axis_index("s")
    wid = s * NC + c
    pltpu.sync_copy(edges_hbm.at[0, pl.ds(wid * DNCH, DNCH)], idx_s)
    pltpu.sync_copy(edges_hbm.at[1, pl.ds(wid * DNCH, DNCH)], idx_d)
    zeros16 = jnp.zeros((LANES,), _F32)

    def zbody(i, carry):
        degs[pl.ds(i * LANES, LANES)] = zeros16
        degd[pl.ds(i * LANES, LANES)] = zeros16
        return carry

    lax.fori_loop(0, N // LANES, zbody, 0)
    ones16 = jnp.ones((LANES,), _F32)

    def hbody(i, carry):
        for j in range(DCH // LANES):
            v = idx_s[i, pl.ds(j * LANES, LANES)]
            plsc.addupdate_scatter(degs, [v], ones16)
            w = idx_d[i, pl.ds(j * LANES, LANES)]
            plsc.addupdate_scatter(degd, [w], ones16)
        return carry

    lax.fori_loop(0, DNCH, hbody, 0)
    pltpu.sync_copy(degs, out_hbm.at[wid, 0])
    pltpu.sync_copy(degd, out_hbm.at[wid, 1])


@functools.cache
def _deg():
    mesh = plsc.VectorSubcoreMesh(core_axis_name="c", subcore_axis_name="s",
                                  num_cores=NC, num_subcores=NS)
    return pl.kernel(
        _deg_body,
        out_type=jax.ShapeDtypeStruct((NW, 2, N), _F32),
        mesh=mesh,
        compiler_params=pltpu.CompilerParams(use_tc_tiling_on_sc=False,
                                             needs_layout_passes=False),
        scratch_types=[
            pltpu.VMEM((DNCH, DCH), jnp.int32),
            pltpu.VMEM((DNCH, DCH), jnp.int32),
            pltpu.VMEM((N,), _F32),
            pltpu.VMEM((N,), _F32),
        ],
    )


def _mp_body(t_hbm, edges_hbm, out_hbm, agg_sh, idx_s, idx_d, rows,
             gsem, ssem):
    c = lax.axis_index("c")
    s = lax.axis_index("s")
    tc = t_hbm.at[c]
    pltpu.sync_copy(edges_hbm.at[0, pl.ds(s * NCH, NCH)], idx_s)
    pltpu.sync_copy(edges_hbm.at[1, pl.ds(s * NCH, NCH)], idx_d)
    zeros32 = jnp.zeros((2 * LANES,), _BF16)

    def zb(i, carry):
        for j in range(HH // (2 * LANES)):
            rows[0, i, pl.ds(j * 2 * LANES, 2 * LANES)] = zeros32
        return carry

    lax.fori_loop(0, ZCH, zb, 0)
    for k in range(NZ):
        pltpu.sync_copy(rows.at[0], agg_sh.at[pl.ds(s * RPT + k * ZCH, ZCH)])
    plsc.subcore_barrier()

    def g_issue(j, b):
        pltpu.async_copy(tc.at[idx_s.at[j]], rows.at[b], gsem.at[b])

    def g_wait(b):
        pltpu.make_async_copy(tc.at[idx_s.at[0]], rows.at[b],
                              gsem.at[b]).wait()

    def s_issue(j, b):
        pltpu.async_copy(rows.at[b], agg_sh.at[idx_d.at[j]], ssem.at[b],
                         add=True)

    def s_wait(b):
        pltpu.make_async_copy(rows.at[b], agg_sh.at[idx_d.at[0]],
                              ssem.at[b]).wait()

    for b in range(R - 1):
        g_issue(b, b)
    for b in range(R):
        g_wait(b)
        s_issue(b, b)
        if b >= 1:
            s_wait((b - 1) % R)
        g_issue(b + R - 1, (b - 1) % R)

    def grp(g, carry):
        for b in range(R):
            j = g * R + b
            g_wait(b)
            s_issue(j, b)
            s_wait((b - 1) % R)
            g_issue(j + R - 1, (b - 1) % R)
        return carry

    lax.fori_loop(1, NG - 1, grp, 0)
    for b in range(R):
        j = (NG - 1) * R + b
        g_wait(b)
        s_issue(j, b)
        if b == 0:
            s_wait((b - 1) % R)
            g_issue(j + R - 1, (b - 1) % R)
    for b in range(R):
        s_wait(b)

    plsc.subcore_barrier()
    for k in range(NZ):
        pltpu.sync_copy(agg_sh.at[pl.ds(s * RPT + k * ZCH, ZCH)], rows.at[0])
        pltpu.sync_copy(rows.at[0],
                        out_hbm.at[c, pl.ds(s * RPT + k * ZCH, ZCH)])


@functools.cache
def _mp():
    mesh = plsc.VectorSubcoreMesh(core_axis_name="c", subcore_axis_name="s",
                                  num_cores=NC, num_subcores=NS)
    return pl.kernel(
        _mp_body,
        out_type=jax.ShapeDtypeStruct((NC, N, HH), _BF16),
        mesh=mesh,
        compiler_params=pltpu.CompilerParams(use_tc_tiling_on_sc=False,
                                             needs_layout_passes=False),
        scratch_types=[
            pltpu.VMEM_SHARED((N, HH), _BF16),
            pltpu.VMEM((NCH, CH), jnp.int32),
            pltpu.VMEM((NCH, CH), jnp.int32),
            pltpu.VMEM((R, CH, HH), _BF16),
            pltpu.SemaphoreType.DMA((R,)),
            pltpu.SemaphoreType.DMA((R,)),
        ],
    )


def _norm_body(degp_ref, norm_ref):
    acc = degp_ref[0]
    for i in range(1, NW):
        acc = acc + degp_ref[i]
    norm_ref[...] = lax.rsqrt(jnp.maximum(acc, 1.0))


_norms = pl.pallas_call(
    _norm_body,
    out_shape=jax.ShapeDtypeStruct((2, N), _F32),
)


def _mm1_body(x_ref, w_ref, nrm_ref, o_ref):
    ns = lax.transpose(nrm_ref[0:1, :], (1, 0))
    for c in range(NC):
        t = jnp.dot(x_ref[...], w_ref[c],
                    preferred_element_type=_F32) * ns
        o_ref[c] = t.astype(_BF16)


_mm1 = pl.pallas_call(
    _mm1_body,
    out_shape=jax.ShapeDtypeStruct((NC, N, HH), _BF16),
)


def _tcb_body(p_ref, nrm_ref, b1_ref, w2_ref, o_ref):
    ns = lax.transpose(nrm_ref[0:1, :], (1, 0))
    nd = lax.transpose(nrm_ref[1:2, :], (1, 0))
    h0 = jnp.maximum(p_ref[0].astype(_F32) * nd + b1_ref[0], 0.0)
    h1 = jnp.maximum(p_ref[1].astype(_F32) * nd + b1_ref[1], 0.0)
    for c in range(NC):
        t2 = (jnp.dot(h0, w2_ref[0, c], preferred_element_type=_F32)
              + jnp.dot(h1, w2_ref[1, c], preferred_element_type=_F32))
        o_ref[c] = (t2 * ns).astype(_BF16)


_tcb = pl.pallas_call(
    _tcb_body,
    out_shape=jax.ShapeDtypeStruct((NC, N, HH), _BF16),
)


def _tcc_body(q_ref, nrm_ref, b2_ref, wc1_ref, bc1_ref, wc2_ref, bc2_ref,
              wc3_ref, bc3_ref, o_ref):
    nd = lax.transpose(nrm_ref[1:2, :], (1, 0))
    hg = []
    for c in range(NC):
        h2 = jnp.maximum(q_ref[c].astype(_F32) * nd + b2_ref[c], 0.0)
        hg.append(jnp.mean(h2, axis=0, keepdims=True))
    o1 = jnp.maximum(jnp.dot(hg[0], wc1_ref[0], preferred_element_type=_F32,
                             precision=_HI)
                     + jnp.dot(hg[1], wc1_ref[1], preferred_element_type=_F32,
                               precision=_HI)
                     + bc1_ref[...], 0.0)
    o2 = jnp.maximum(jnp.dot(o1, wc2_ref[...], preferred_element_type=_F32,
                             precision=_HI) + bc2_ref[...], 0.0)
    o_ref[...] = jnp.dot(o2, wc3_ref[...], preferred_element_type=_F32,
                         precision=_HI) + bc3_ref[...]


_tcc = pl.pallas_call(
    _tcc_body,
    out_shape=jax.ShapeDtypeStruct((1, C), _F32),
)


def _split_cols(w):
    return jnp.stack([w[:, :HH], w[:, HH:]])


def _split_rows(w):
    return jnp.stack([w[:HH, :], w[HH:, :]])


def kernel(x, edge_index, W1, b1, W2, b2, Wc1, bc1, Wc2, bc2, Wc3, bc3):
    edges_d = edge_index.reshape(2, E // DCH, DCH)
    edges_m = edge_index.reshape(2, E // CH, CH)
    degp = _deg()(edges_d)
    norms = _norms(degp)
    w1s = _split_cols(W1)
    w2q = jnp.stack([_split_cols(W2[:HH]), _split_cols(W2[HH:])])
    b1s = b1.reshape(2, 1, HH)
    b2s = b2.reshape(2, 1, HH)
    wc1s = _split_rows(Wc1)
    t1 = _mm1(x, w1s, norms)
    p = _mp()(t1, edges_m)
    t2 = _tcb(p, norms, b1s, w2q)
    q = _mp()(t2, edges_m)
    o = _tcc(q, norms, b2s, wc1s, bc1.reshape(1, H), Wc2, bc2.reshape(1, H),
             Wc3, bc3.reshape(1, C))
    return o
